# baseline (speedup 1.0000x reference)
import functools

import jax
import jax.numpy as jnp
from jax import lax
from jax.experimental import pallas as pl
from jax.experimental.pallas import tpu as pltpu
from jax.experimental.pallas import tpu_sc as plsc

EMBED = 64
PAD_W = 128
NUM_CORES = 2
NUM_SUBCORES = 16
NUM_WORKERS = NUM_CORES * NUM_SUBCORES
CHUNK = 128
PASS_ROWS = 256


def kernel(author_idx, publisher_idx, year_idx, author_table,
           publisher_table, year_table):
    batch = author_idx.shape[0]
    b_per_w = batch // NUM_WORKERS
    n_pass = b_per_w // PASS_ROWS
    chunks_per_pass = PASS_ROWS // CHUNK

    a_idx = author_idx.astype(jnp.int32)
    p_idx = publisher_idx.astype(jnp.int32)
    y_idx = year_idx.astype(jnp.int32)

    def padt(t):
        return jnp.pad(t, ((0, 0), (0, PAD_W - EMBED)))

    a_t = padt(author_table)
    p_t = padt(publisher_table)
    y_t = padt(year_table)

    mesh = plsc.VectorSubcoreMesh(core_axis_name="c", subcore_axis_name="s")

    @functools.partial(
        pl.kernel,
        mesh=mesh,
        out_type=jax.ShapeDtypeStruct((batch, 3 * EMBED), jnp.float32),
        scratch_types=[
            pltpu.VMEM((b_per_w,), jnp.int32),
            pltpu.VMEM((b_per_w,), jnp.int32),
            pltpu.VMEM((b_per_w,), jnp.int32),
            pltpu.VMEM((PASS_ROWS, PAD_W), jnp.float32),
            pltpu.VMEM((PASS_ROWS, PAD_W), jnp.float32),
            pltpu.VMEM((PASS_ROWS, PAD_W), jnp.float32),
            pltpu.SemaphoreType.DMA,
        ],
        compiler_params=pltpu.CompilerParams(use_tc_tiling_on_sc=False),
    )
    def sc_gather3(a_hbm, p_hbm, y_hbm, ai, pi, yi, out,
                   ai_v, pi_v, yi_v, ar_v, pr_v, yr_v, sem):
        wid = lax.axis_index("s") * NUM_CORES + lax.axis_index("c")
        base = wid * b_per_w
        pltpu.sync_copy(ai.at[pl.ds(base, b_per_w)], ai_v)
        pltpu.sync_copy(pi.at[pl.ds(base, b_per_w)], pi_v)
        pltpu.sync_copy(yi.at[pl.ds(base, b_per_w)], yi_v)
        for p in range(n_pass):
            copies = []
            for j in range(chunks_per_pass):
                src = pl.ds(p * PASS_ROWS + j * CHUNK, CHUNK)
                dst = pl.ds(j * CHUNK, CHUNK)
                copies.append(pltpu.async_copy(
                    a_hbm.at[ai_v.at[src]], ar_v.at[dst], sem))
                copies.append(pltpu.async_copy(
                    p_hbm.at[pi_v.at[src]], pr_v.at[dst], sem))
                copies.append(pltpu.async_copy(
                    y_hbm.at[yi_v.at[src]], yr_v.at[dst], sem))
            for c in copies:
                c.wait()
            rows = pl.ds(base + p * PASS_ROWS, PASS_ROWS)
            pltpu.sync_copy(ar_v.at[:, pl.ds(0, EMBED)],
                            out.at[rows, pl.ds(0, EMBED)])
            pltpu.sync_copy(pr_v.at[:, pl.ds(0, EMBED)],
                            out.at[rows, pl.ds(EMBED, EMBED)])
            pltpu.sync_copy(yr_v.at[:, pl.ds(0, EMBED)],
                            out.at[rows, pl.ds(2 * EMBED, EMBED)])

    return sc_gather3(a_t, p_t, y_t, a_idx, p_idx, y_idx)


# speedup vs baseline: 1.1087x; 1.1087x over previous
import functools

import jax
import jax.numpy as jnp
from jax import lax
from jax.experimental import pallas as pl
from jax.experimental.pallas import tpu as pltpu
from jax.experimental.pallas import tpu_sc as plsc

EMBED = 64
NUM_CORES = 2
NUM_SUBCORES = 16
NUM_WORKERS = NUM_CORES * NUM_SUBCORES
CHUNK = 128


def kernel(author_idx, publisher_idx, year_idx, author_table,
           publisher_table, year_table):
    batch = author_idx.shape[0]
    b_per_w = batch // NUM_WORKERS
    n_chunks = b_per_w // CHUNK

    a_idx = author_idx.astype(jnp.int32)
    p_idx = publisher_idx.astype(jnp.int32)
    y_idx = year_idx.astype(jnp.int32)

    mesh = plsc.VectorSubcoreMesh(core_axis_name="c", subcore_axis_name="s")

    @functools.partial(
        pl.kernel,
        mesh=mesh,
        out_type=jax.ShapeDtypeStruct((batch, 3 * EMBED), jnp.float32),
        scratch_types=[
            pltpu.VMEM((b_per_w,), jnp.int32),
            pltpu.VMEM((b_per_w,), jnp.int32),
            pltpu.VMEM((b_per_w,), jnp.int32),
            pltpu.VMEM((b_per_w, EMBED), jnp.float32),
            pltpu.VMEM((b_per_w, EMBED), jnp.float32),
            pltpu.VMEM((b_per_w, EMBED), jnp.float32),
            pltpu.SemaphoreType.DMA,
        ],
        compiler_params=pltpu.CompilerParams(use_tc_tiling_on_sc=False),
    )
    def sc_gather3(a_hbm, p_hbm, y_hbm, ai, pi, yi, out,
                   ai_v, pi_v, yi_v, ar_v, pr_v, yr_v, sem):
        wid = lax.axis_index("s") * NUM_CORES + lax.axis_index("c")
        base = wid * b_per_w
        pltpu.sync_copy(ai.at[pl.ds(base, b_per_w)], ai_v)
        pltpu.sync_copy(pi.at[pl.ds(base, b_per_w)], pi_v)
        pltpu.sync_copy(yi.at[pl.ds(base, b_per_w)], yi_v)
        copies = []
        for j in range(n_chunks):
            rows = pl.ds(j * CHUNK, CHUNK)
            copies.append(pltpu.async_copy(
                a_hbm.at[ai_v.at[rows]], ar_v.at[rows], sem))
            copies.append(pltpu.async_copy(
                p_hbm.at[pi_v.at[rows]], pr_v.at[rows], sem))
            copies.append(pltpu.async_copy(
                y_hbm.at[yi_v.at[rows]], yr_v.at[rows], sem))
        for c in copies:
            c.wait()
        dst = pl.ds(base, b_per_w)
        pltpu.sync_copy(ar_v, out.at[dst, pl.ds(0, EMBED)])
        pltpu.sync_copy(pr_v, out.at[dst, pl.ds(EMBED, EMBED)])
        pltpu.sync_copy(yr_v, out.at[dst, pl.ds(2 * EMBED, EMBED)])

    return sc_gather3(author_table, publisher_table, year_table,
                      a_idx, p_idx, y_idx)
